# baseline (speedup 1.0000x reference)
import jax
import jax.numpy as jnp
from jax import lax
from jax.experimental import pallas as pl


def _conv(x, w, b, stride, pad):
    y = lax.conv_general_dilated(
        x, w, (stride, stride), [(pad, pad), (pad, pad)],
        dimension_numbers=('NCHW', 'OIHW', 'NCHW'))
    return y + b[None, :, None, None]


def _convT(x, w, b, stride, pad):
    kh, kw = w.shape[2], w.shape[3]
    w2 = jnp.transpose(w[:, :, ::-1, ::-1], (1, 0, 2, 3))
    y = lax.conv_general_dilated(
        x, w2, (1, 1),
        [(kh - 1 - pad, kh - 1 - pad), (kw - 1 - pad, kw - 1 - pad)],
        lhs_dilation=(stride, stride),
        dimension_numbers=('NCHW', 'OIHW', 'NCHW'))
    return y + b[None, :, None, None]


def _bn(x, g, b, eps=1e-5):
    m = jnp.mean(x, axis=(0, 2, 3), keepdims=True)
    v = jnp.var(x, axis=(0, 2, 3), keepdims=True)
    return (x - m) / jnp.sqrt(v + eps) * g[None, :, None, None] + b[None, :, None, None]


_H = 224
_CI = 128
_CO = 3
_TH = 32


def _bnconv3_body(cur_ref, nxt_ref, sc_ref, sh_ref, w_ref, b_ref, out_ref):
    raw = jnp.concatenate([cur_ref[0], nxt_ref[0, :, 0]], axis=1)
    sc = sc_ref[...][:, None, None]
    sh = sh_ref[...][:, None, None]
    win = jnp.maximum(raw * sc + sh, 0.0)
    acc = jnp.zeros((_CO, _TH, _H), jnp.float32)
    for ky in range(3):
        xk = win[:, ky:ky + _TH, :].reshape(_CI, _TH * (_H + 2))
        for kx in range(3):
            wt = w_ref[:, :, ky, kx]
            p = lax.dot_general(
                wt, xk,
                dimension_numbers=(((1,), (0,)), ((), ())),
                preferred_element_type=jnp.float32)
            acc += p.reshape(_CO, _TH, _H + 2)[:, :, kx:kx + _H]
    out = acc + b_ref[...][:, None, None]
    out_ref[...] = out[None]


def _bnconv3_pallas(y2, scale, shift, w3, b3):
    w2 = jnp.transpose(w3[:, :, ::-1, ::-1], (1, 0, 2, 3))
    hp = jnp.pad(y2, ((0, 0), (0, 0), (1, 1), (1, 1)), constant_values=-1e30)
    nst = _H // _TH
    halo = jnp.stack([hp[:, :, _TH::_TH], hp[:, :, _TH + 1::_TH]], axis=3)
    return pl.pallas_call(
        _bnconv3_body,
        grid=(y2.shape[0], nst),
        in_specs=[
            pl.BlockSpec((1, _CI, _TH, _H + 2), lambda b, i: (b, 0, i, 0)),
            pl.BlockSpec((1, _CI, 1, 2, _H + 2), lambda b, i: (b, 0, i, 0, 0)),
            pl.BlockSpec((_CI,), lambda b, i: (0,)),
            pl.BlockSpec((_CI,), lambda b, i: (0,)),
            pl.BlockSpec((_CO, _CI, 3, 3), lambda b, i: (0, 0, 0, 0)),
            pl.BlockSpec((_CO,), lambda b, i: (0,)),
        ],
        out_specs=pl.BlockSpec((1, _CO, _TH, _H), lambda b, i: (b, 0, i, 0)),
        out_shape=jax.ShapeDtypeStruct((y2.shape[0], _CO, _H, _H), jnp.float32),
    )(hp, halo, scale, shift, w2, b3)


def kernel(x, ew1, eb1, eg1, eB1, ew2, eb2, eg2, eB2, ew3, eb3, codebook,
           dw1, db1, dg1, dB1, dw2, db2, dg2, dB2, dw3, db3,
           commitment_cost=0.25):
    h = jax.nn.relu(_bn(_conv(x, ew1, eb1, 2, 1), eg1, eB1))
    h = jax.nn.relu(_bn(_conv(h, ew2, eb2, 2, 1), eg2, eB2))
    z = _conv(h, ew3, eb3, 1, 1)
    zp = jnp.transpose(z, (0, 2, 3, 1))
    D = zp.shape[-1]
    flat = zp.reshape(-1, D)
    dist = jnp.sum(flat ** 2, axis=1, keepdims=True) + jnp.sum(codebook ** 2, axis=1) - 2.0 * (flat @ codebook.T)
    idx = jnp.argmin(dist, axis=1)
    quant = jnp.take(codebook, idx, axis=0).reshape(zp.shape)
    quant = jnp.transpose(quant, (0, 3, 1, 2))
    e_loss = jnp.mean((jax.lax.stop_gradient(quant) - z) ** 2)
    q_loss = jnp.mean((quant - jax.lax.stop_gradient(z)) ** 2)
    loss = q_loss + commitment_cost * e_loss
    quant_st = z + jax.lax.stop_gradient(quant - z)
    h1 = jax.nn.relu(_bn(_convT(quant_st, dw1, db1, 2, 1), dg1, dB1))

    y2 = _convT(h1, dw2, db2, 2, 1)
    m2 = jnp.mean(y2, axis=(0, 2, 3))
    v2 = jnp.var(y2, axis=(0, 2, 3))
    scale2 = dg2 / jnp.sqrt(v2 + 1e-5)
    shift2 = dB2 - m2 * scale2
    recon = _bnconv3_pallas(y2, scale2, shift2, dw3, db3)
    return recon, loss
